# Initial kernel scaffold; baseline (speedup 1.0000x reference)
import jax
import jax.numpy as jnp
from jax.experimental import pallas as pl


def kernel(x, edge_index, edge_weight, batch, device, W1_0, b1_0, W2_0, b2_0, g0, be0, W1_1, b1_1, W2_1, b2_1, g1, be1, fc_W, fc_b, fc1_W, fc1_b, fc2_W, fc2_b):
    raise NotImplementedError("write your pallas kernel here")


# speedup vs baseline: 13.8600x; 13.8600x over previous
import functools

import jax
import jax.numpy as jnp
from jax import lax
from jax.experimental import pallas as pl
from jax.experimental.pallas import tpu as pltpu
from jax.experimental.pallas import tpu_sc as plsc

N = 10000
E = 320000
F_IN = 128
DIM = 64
C = 10
G = 256

NC = 2
NS = 16
NW = NC * NS
EPW = E // NW
CH = 400
NCH = EPW // CH
NPAD = 10240
RPT = NPAD // NS


@functools.lru_cache(maxsize=None)
def _get_edge_pass():
    mesh = plsc.VectorSubcoreMesh(core_axis_name="c", subcore_axis_name="s")
    return functools.partial(
        pl.kernel,
        out_type=jax.ShapeDtypeStruct((NC * NPAD, DIM), jnp.float32),
        mesh=mesh,
        scratch_types=[
            pltpu.VMEM((CH,), jnp.int32),
            pltpu.VMEM((CH,), jnp.int32),
            pltpu.VMEM((CH,), jnp.int32),
            pltpu.VMEM((CH,), jnp.int32),
            pltpu.VMEM((CH,), jnp.float32),
            pltpu.VMEM((CH,), jnp.float32),
            pltpu.VMEM((CH, DIM), jnp.float32),
            pltpu.VMEM((CH, DIM), jnp.float32),
            pltpu.VMEM_SHARED((NPAD, DIM), jnp.float32),
            pltpu.SemaphoreType.DMA,
            pltpu.SemaphoreType.DMA,
            pltpu.SemaphoreType.DMA,
            pltpu.SemaphoreType.DMA,
        ],
        compiler_params=pltpu.CompilerParams(use_tc_tiling_on_sc=False),
    )(_edge_pass_body)


def _edge_pass_body(u_hbm, ei_hbm, w_hbm, zeros_hbm, out_hbm,
                    src0, src1, dst0, dst1, w0, w1, rows0, rows1,
                    acc, isem0, isem1, gsem0, gsem1):
    cid = lax.axis_index("c")
    sid = lax.axis_index("s")
    r0 = sid * RPT
    pltpu.sync_copy(zeros_hbm.at[pl.ds(r0, RPT)], acc.at[pl.ds(r0, RPT)])
    plsc.subcore_barrier()

    base = (cid * NS + sid) * EPW
    srcb, dstb, wb = [src0, src1], [dst0, dst1], [w0, w1]
    rowsb, isem, gsem = [rows0, rows1], [isem0, isem1], [gsem0, gsem1]

    def start_idx(c, b):
        off = base + c * CH
        pltpu.async_copy(ei_hbm.at[0, pl.ds(off, CH)], srcb[b], isem[b])
        pltpu.async_copy(ei_hbm.at[1, pl.ds(off, CH)], dstb[b], isem[b])
        pltpu.async_copy(w_hbm.at[pl.ds(off, CH)], wb[b], isem[b])

    def wait_idx(c, b):
        off = base + c * CH
        pltpu.make_async_copy(ei_hbm.at[0, pl.ds(off, CH)], srcb[b], isem[b]).wait()
        pltpu.make_async_copy(ei_hbm.at[1, pl.ds(off, CH)], dstb[b], isem[b]).wait()
        pltpu.make_async_copy(w_hbm.at[pl.ds(off, CH)], wb[b], isem[b]).wait()

    def start_gather(b):
        pltpu.async_copy(u_hbm.at[srcb[b]], rowsb[b], gsem[b])

    def wait_gather(b):
        pltpu.make_async_copy(u_hbm.at[srcb[b]], rowsb[b], gsem[b]).wait()

    def compute(b):
        rows, wv = rowsb[b], wb[b]

        @plsc.parallel_loop(0, CH // 16, 1, unroll=2)
        def g_body(g):
            w16 = wv[pl.ds(g * 16, 16)]
            for j in range(16):
                e = g * 16 + j
                wj = jnp.take(w16, jnp.full((16,), j, jnp.int32))
                for q in range(DIM // 16):
                    rows[e, pl.ds(q * 16, 16)] = rows[e, pl.ds(q * 16, 16)] * wj

        pltpu.sync_copy(rows, acc.at[dstb[b]], add=True)

    start_idx(0, 0)
    wait_idx(0, 0)
    start_gather(0)
    start_idx(1, 1)

    def pair_body(t, carry):
        for p in (1, 2):
            c = 2 * t + p
            b, nb = p % 2, (p + 1) % 2
            wait_idx(c, b)
            start_gather(b)
            wait_gather(nb)
            compute(nb)
            start_idx(c + 1, nb)
        return carry

    n_pairs = (NCH - 2) // 2
    lax.fori_loop(0, n_pairs, pair_body, 0)
    for c in range(2 * n_pairs + 1, NCH):
        b, nb = c % 2, (c + 1) % 2
        wait_idx(c, b)
        start_gather(b)
        wait_gather(nb)
        compute(nb)
        if c + 1 < NCH:
            start_idx(c + 1, nb)
    b = (NCH - 1) % 2
    wait_gather(b)
    compute(b)

    plsc.subcore_barrier()
    pltpu.sync_copy(acc.at[pl.ds(r0, RPT)],
                    out_hbm.at[pl.ds(cid * NPAD + r0, RPT)])


def _dot_t(a, w):
    return lax.dot_general(a, w, (((1,), (1,)), ((), ())),
                           preferred_element_type=jnp.float32)


def _mm_body(x_ref, w_ref, o_ref):
    o_ref[...] = _dot_t(x_ref[...], w_ref[...])


def _mid_body(u_ref, a_ref, b1_ref, w2_ref, b2_ref, g_ref, be_ref,
              w1n_ref, h_ref, un_ref):
    z = u_ref[...] + a_ref[0:N] + a_ref[NPAD:NPAD + N] + b1_ref[...]
    z = jnp.maximum(z, 0.0)
    h = jnp.maximum(_dot_t(z, w2_ref[...]) + b2_ref[...], 0.0)
    m = jnp.mean(h, axis=0, keepdims=True)
    v = jnp.mean(h * h, axis=0, keepdims=True) - m * m
    hn = g_ref[...] * (h - m) * lax.rsqrt(v + 1e-5) + be_ref[...]
    h_ref[...] = hn
    un_ref[...] = _dot_t(hn, w1n_ref[...])


def _fin_body(u_ref, a_ref, b1_ref, w2_ref, b2_ref, g_ref, be_ref,
              h1_ref, batch_ref, fcw_ref, fcb_ref, fc1w_ref, fc1b_ref,
              fc2w_ref, fc2b_ref, o_ref):
    z = u_ref[...] + a_ref[0:N] + a_ref[NPAD:NPAD + N] + b1_ref[...]
    z = jnp.maximum(z, 0.0)
    h2 = jnp.maximum(_dot_t(z, w2_ref[...]) + b2_ref[...], 0.0)
    m = jnp.mean(h2, axis=0, keepdims=True)
    v = jnp.mean(h2 * h2, axis=0, keepdims=True) - m * m
    h2 = g_ref[...] * (h2 - m) * lax.rsqrt(v + 1e-5) + be_ref[...]
    gid = lax.broadcasted_iota(jnp.int32, (G, N), 0)
    seg = jnp.where(gid == batch_ref[...], 1.0, 0.0)
    p1 = jnp.dot(seg, h1_ref[...], preferred_element_type=jnp.float32)
    p2 = jnp.dot(seg, h2, preferred_element_type=jnp.float32)
    fcw = fcw_ref[...]
    zz = jnp.maximum(_dot_t(p1, fcw[:, :DIM]) + _dot_t(p2, fcw[:, DIM:])
                     + fcb_ref[...], 0.0)
    zz = jnp.maximum(_dot_t(zz, fc1w_ref[...]) + fc1b_ref[...], 0.0)
    zz = _dot_t(zz, fc2w_ref[...]) + fc2b_ref[...]
    zz = zz - jnp.max(zz, axis=-1, keepdims=True)
    o_ref[...] = zz - jnp.log(jnp.sum(jnp.exp(zz), axis=-1, keepdims=True))


def kernel(x, edge_index, edge_weight, batch, device,
           W1_0, b1_0, W2_0, b2_0, g0, be0,
           W1_1, b1_1, W2_1, b2_1, g1, be1,
           fc_W, fc_b, fc1_W, fc1_b, fc2_W, fc2_b):
    zeros = jnp.zeros((NPAD, DIM), jnp.float32)

    u0 = pl.pallas_call(
        _mm_body,
        out_shape=jax.ShapeDtypeStruct((N, DIM), jnp.float32),
    )(x, W1_0)

    edge_pass = _get_edge_pass()
    a0 = edge_pass(u0, edge_index, edge_weight, zeros)

    h1, u1 = pl.pallas_call(
        _mid_body,
        out_shape=(jax.ShapeDtypeStruct((N, DIM), jnp.float32),
                   jax.ShapeDtypeStruct((N, DIM), jnp.float32)),
    )(u0, a0, b1_0.reshape(1, DIM), W2_0, b2_0.reshape(1, DIM),
      g0.reshape(1, DIM), be0.reshape(1, DIM), W1_1)

    a1 = edge_pass(u1, edge_index, edge_weight, zeros)

    out = pl.pallas_call(
        _fin_body,
        out_shape=jax.ShapeDtypeStruct((G, C), jnp.float32),
    )(u1, a1, b1_1.reshape(1, DIM), W2_1, b2_1.reshape(1, DIM),
      g1.reshape(1, DIM), be1.reshape(1, DIM), h1,
      batch.reshape(1, N), fc_W, fc_b.reshape(1, DIM),
      fc1_W, fc1_b.reshape(1, DIM), fc2_W, fc2_b.reshape(1, C))
    return out


# speedup vs baseline: 15.3306x; 1.1061x over previous
import functools

import jax
import jax.numpy as jnp
from jax import lax
from jax.experimental import pallas as pl
from jax.experimental.pallas import tpu as pltpu
from jax.experimental.pallas import tpu_sc as plsc

N = 10000
E = 320000
F_IN = 128
DIM = 64
C = 10
G = 256

NC = 2
NS = 16
NW = NC * NS
EPW = E // NW
CH = 400
NCH = EPW // CH
NPAD = 10240
RPT = NPAD // NS


@functools.lru_cache(maxsize=None)
def _get_edge_pass():
    mesh = plsc.VectorSubcoreMesh(core_axis_name="c", subcore_axis_name="s")
    return functools.partial(
        pl.kernel,
        out_type=jax.ShapeDtypeStruct((NC * NPAD, DIM), jnp.float32),
        mesh=mesh,
        scratch_types=[
            pltpu.VMEM((CH,), jnp.int32),
            pltpu.VMEM((CH,), jnp.int32),
            pltpu.VMEM((CH,), jnp.int32),
            pltpu.VMEM((CH,), jnp.int32),
            pltpu.VMEM((CH,), jnp.float32),
            pltpu.VMEM((CH,), jnp.float32),
            pltpu.VMEM((CH, DIM), jnp.float32),
            pltpu.VMEM((CH, DIM), jnp.float32),
            pltpu.VMEM((CH,), jnp.int32),
            pltpu.VMEM((CH,), jnp.int32),
            pltpu.VMEM_SHARED((NPAD, DIM), jnp.float32),
            pltpu.SemaphoreType.DMA,
            pltpu.SemaphoreType.DMA,
            pltpu.SemaphoreType.DMA,
            pltpu.SemaphoreType.DMA,
            pltpu.SemaphoreType.DMA,
            pltpu.SemaphoreType.DMA,
        ],
        compiler_params=pltpu.CompilerParams(use_tc_tiling_on_sc=False),
    )(_edge_pass_body)


def _edge_pass_body(u_hbm, ei_hbm, w_hbm, zeros_hbm, out_hbm,
                    src0, src1, dst0, dst1, w0, w1, rows0, rows1,
                    sdst0, sdst1, acc,
                    isem0, isem1, gsem0, gsem1, ssem0, ssem1):
    cid = lax.axis_index("c")
    sid = lax.axis_index("s")
    r0 = sid * RPT
    pltpu.sync_copy(zeros_hbm.at[pl.ds(r0, RPT)], acc.at[pl.ds(r0, RPT)])
    plsc.subcore_barrier()

    base = (cid * NS + sid) * EPW
    srcb, dstb, wb = [src0, src1], [dst0, dst1], [w0, w1]
    rowsb, isem, gsem = [rows0, rows1], [isem0, isem1], [gsem0, gsem1]
    sdstb, ssem = [sdst0, sdst1], [ssem0, ssem1]

    def start_idx(c, b):
        off = base + c * CH
        pltpu.async_copy(ei_hbm.at[0, pl.ds(off, CH)], srcb[b], isem[b])
        pltpu.async_copy(ei_hbm.at[1, pl.ds(off, CH)], dstb[b], isem[b])
        pltpu.async_copy(w_hbm.at[pl.ds(off, CH)], wb[b], isem[b])

    def wait_idx(c, b):
        off = base + c * CH
        pltpu.make_async_copy(ei_hbm.at[0, pl.ds(off, CH)], srcb[b], isem[b]).wait()
        pltpu.make_async_copy(ei_hbm.at[1, pl.ds(off, CH)], dstb[b], isem[b]).wait()
        pltpu.make_async_copy(w_hbm.at[pl.ds(off, CH)], wb[b], isem[b]).wait()

    def start_gather(b):
        pltpu.async_copy(u_hbm.at[srcb[b]], rowsb[b], gsem[b])

    def wait_gather(b):
        pltpu.make_async_copy(u_hbm.at[srcb[b]], rowsb[b], gsem[b]).wait()

    def wait_scatter(b):
        pltpu.make_async_copy(rowsb[b], acc.at[sdstb[b]], ssem[b]).wait()

    def compute(b):
        rows, wv = rowsb[b], wb[b]

        @plsc.parallel_loop(0, CH // 16, 1, unroll=2)
        def d_body(d):
            sdstb[b][pl.ds(d * 16, 16)] = dstb[b][pl.ds(d * 16, 16)]

        @plsc.parallel_loop(0, CH // 16, 1, unroll=2)
        def g_body(g):
            w16 = wv[pl.ds(g * 16, 16)]
            for j in range(16):
                e = g * 16 + j
                wj = jnp.take(w16, jnp.full((16,), j, jnp.int32))
                for q in range(DIM // 16):
                    rows[e, pl.ds(q * 16, 16)] = rows[e, pl.ds(q * 16, 16)] * wj

        pltpu.async_copy(rows, acc.at[sdstb[b]], ssem[b], add=True)

    def step(c, b, nb, first_on_parity=False, prefetch=True, guard=False):
        wait_idx(c, b)
        if not first_on_parity:
            wait_scatter(b)
        start_gather(b)
        wait_gather(nb)
        compute(nb)
        if prefetch:
            if guard:
                @pl.when(c + 1 < NCH)
                def _():
                    start_idx(c + 1, nb)
            else:
                start_idx(c + 1, nb)

    start_idx(0, 0)
    wait_idx(0, 0)
    start_gather(0)
    start_idx(1, 1)
    step(1, 1, 0, first_on_parity=True)
    step(2, 0, 1, first_on_parity=True)

    def pair_body(t, carry):
        for p in (1, 2):
            step(2 * t + 2 + p, p % 2, (p + 1) % 2, guard=True)
        return carry

    n_pairs = (NCH - 3) // 2
    lax.fori_loop(0, n_pairs, pair_body, 0)
    for c in range(3 + 2 * n_pairs, NCH):
        step(c, c % 2, (c + 1) % 2, prefetch=(c + 1 < NCH))
    b = (NCH - 1) % 2
    wait_gather(b)
    compute(b)
    wait_scatter(0)
    wait_scatter(1)

    plsc.subcore_barrier()
    pltpu.sync_copy(acc.at[pl.ds(r0, RPT)],
                    out_hbm.at[pl.ds(cid * NPAD + r0, RPT)])


def _dot_t(a, w):
    return lax.dot_general(a, w, (((1,), (1,)), ((), ())),
                           preferred_element_type=jnp.float32)


def _mm_body(x_ref, w_ref, o_ref):
    o_ref[...] = _dot_t(x_ref[...], w_ref[...])


def _mid_body(u_ref, a_ref, b1_ref, w2_ref, b2_ref, g_ref, be_ref,
              w1n_ref, h_ref, un_ref):
    z = u_ref[...] + a_ref[0:N] + a_ref[NPAD:NPAD + N] + b1_ref[...]
    z = jnp.maximum(z, 0.0)
    h = jnp.maximum(_dot_t(z, w2_ref[...]) + b2_ref[...], 0.0)
    m = jnp.mean(h, axis=0, keepdims=True)
    v = jnp.mean(h * h, axis=0, keepdims=True) - m * m
    hn = g_ref[...] * (h - m) * lax.rsqrt(v + 1e-5) + be_ref[...]
    h_ref[...] = hn
    un_ref[...] = _dot_t(hn, w1n_ref[...])


def _fin_body(u_ref, a_ref, b1_ref, w2_ref, b2_ref, g_ref, be_ref,
              h1_ref, batch_ref, fcw_ref, fcb_ref, fc1w_ref, fc1b_ref,
              fc2w_ref, fc2b_ref, o_ref):
    z = u_ref[...] + a_ref[0:N] + a_ref[NPAD:NPAD + N] + b1_ref[...]
    z = jnp.maximum(z, 0.0)
    h2 = jnp.maximum(_dot_t(z, w2_ref[...]) + b2_ref[...], 0.0)
    m = jnp.mean(h2, axis=0, keepdims=True)
    v = jnp.mean(h2 * h2, axis=0, keepdims=True) - m * m
    h2 = g_ref[...] * (h2 - m) * lax.rsqrt(v + 1e-5) + be_ref[...]
    gid = lax.broadcasted_iota(jnp.int32, (G, N), 0)
    seg = jnp.where(gid == batch_ref[...], 1.0, 0.0)
    p1 = jnp.dot(seg, h1_ref[...], preferred_element_type=jnp.float32)
    p2 = jnp.dot(seg, h2, preferred_element_type=jnp.float32)
    fcw = fcw_ref[...]
    zz = jnp.maximum(_dot_t(p1, fcw[:, :DIM]) + _dot_t(p2, fcw[:, DIM:])
                     + fcb_ref[...], 0.0)
    zz = jnp.maximum(_dot_t(zz, fc1w_ref[...]) + fc1b_ref[...], 0.0)
    zz = _dot_t(zz, fc2w_ref[...]) + fc2b_ref[...]
    zz = zz - jnp.max(zz, axis=-1, keepdims=True)
    o_ref[...] = zz - jnp.log(jnp.sum(jnp.exp(zz), axis=-1, keepdims=True))


def kernel(x, edge_index, edge_weight, batch, device,
           W1_0, b1_0, W2_0, b2_0, g0, be0,
           W1_1, b1_1, W2_1, b2_1, g1, be1,
           fc_W, fc_b, fc1_W, fc1_b, fc2_W, fc2_b):
    zeros = jnp.zeros((NPAD, DIM), jnp.float32)

    u0 = pl.pallas_call(
        _mm_body,
        out_shape=jax.ShapeDtypeStruct((N, DIM), jnp.float32),
    )(x, W1_0)

    edge_pass = _get_edge_pass()
    a0 = edge_pass(u0, edge_index, edge_weight, zeros)

    h1, u1 = pl.pallas_call(
        _mid_body,
        out_shape=(jax.ShapeDtypeStruct((N, DIM), jnp.float32),
                   jax.ShapeDtypeStruct((N, DIM), jnp.float32)),
    )(u0, a0, b1_0.reshape(1, DIM), W2_0, b2_0.reshape(1, DIM),
      g0.reshape(1, DIM), be0.reshape(1, DIM), W1_1)

    a1 = edge_pass(u1, edge_index, edge_weight, zeros)

    out = pl.pallas_call(
        _fin_body,
        out_shape=jax.ShapeDtypeStruct((G, C), jnp.float32),
    )(u1, a1, b1_1.reshape(1, DIM), W2_1, b2_1.reshape(1, DIM),
      g1.reshape(1, DIM), be1.reshape(1, DIM), h1,
      batch.reshape(1, N), fc_W, fc_b.reshape(1, DIM),
      fc1_W, fc1_b.reshape(1, DIM), fc2_W, fc2_b.reshape(1, C))
    return out
